# initial kernel scaffold (unmeasured)
import jax
import jax.numpy as jnp
from jax import lax
from jax.experimental import pallas as pl
from jax.experimental.pallas import tpu as pltpu

N_DEV = 4


def kernel(x, w_mat, scale_x, scale_w):
    m_per, k = x.shape
    _, n = w_mat.shape
    n_per = n // N_DEV

    def body(x_ref, w_ref, sx_ref, sw_ref, out_ref,
             send_buf, recv_buf, send_sems, recv_sems):
        my = lax.axis_index("i")

        barrier = pltpu.get_barrier_semaphore()
        for d in range(1, N_DEV):
            pl.semaphore_signal(
                barrier, inc=1,
                device_id=((my + d) % N_DEV,),
                device_id_type=pl.DeviceIdType.MESH,
            )
        pl.semaphore_wait(barrier, N_DEV - 1)

        scale = sx_ref[0] * sw_ref[0]
        x_bf = x_ref[:, :].astype(jnp.bfloat16)

        def block_for(dst):
            w_bf = w_ref[:, pl.ds(dst * n_per, n_per)].astype(jnp.bfloat16)
            acc = lax.dot_general(
                x_bf, w_bf, (((1,), (0,)), ((), ())),
                preferred_element_type=jnp.float32,
            )
            y = acc * scale
            return y * (1.0 / (1.0 + jnp.exp(-y)))

        out_ref[pl.ds(my * m_per, m_per), :] = block_for(my)

        for d in range(1, N_DEV):
            dst = (my + d) % N_DEV
            src = (my + (N_DEV - d)) % N_DEV
            send_buf[d - 1, :, :] = block_for(dst)
            rdma = pltpu.make_async_remote_copy(
                src_ref=send_buf.at[d - 1],
                dst_ref=recv_buf.at[d - 1],
                send_sem=send_sems.at[d - 1],
                recv_sem=recv_sems.at[d - 1],
                device_id=(dst,),
                device_id_type=pl.DeviceIdType.MESH,
            )
            rdma.start()
            rdma.wait()
            out_ref[pl.ds(src * m_per, m_per), :] = recv_buf[d - 1, :, :]

    return pl.pallas_call(
        body,
        out_shape=jax.ShapeDtypeStruct((N_DEV * m_per, n_per), jnp.float32),
        in_specs=[
            pl.BlockSpec(memory_space=pltpu.VMEM),
            pl.BlockSpec(memory_space=pltpu.VMEM),
            pl.BlockSpec(memory_space=pltpu.SMEM),
            pl.BlockSpec(memory_space=pltpu.SMEM),
        ],
        out_specs=pl.BlockSpec(memory_space=pltpu.VMEM),
        scratch_shapes=[
            pltpu.VMEM((N_DEV - 1, m_per, n_per), jnp.float32),
            pltpu.VMEM((N_DEV - 1, m_per, n_per), jnp.float32),
            pltpu.SemaphoreType.DMA((N_DEV - 1,)),
            pltpu.SemaphoreType.DMA((N_DEV - 1,)),
        ],
        compiler_params=pltpu.CompilerParams(collective_id=0),
    )(x, w_mat, scale_x, scale_w)


# baseline (device time: 92916 ns/iter reference)
import jax
import jax.numpy as jnp
from jax import lax
from jax.experimental import pallas as pl
from jax.experimental.pallas import tpu as pltpu

N_DEV = 4


def kernel(x, w_mat, scale_x, scale_w):
    m_per, k = x.shape
    _, n = w_mat.shape
    n_per = n // N_DEV

    def body(x_ref, w_ref, sx_ref, sw_ref, out_ref,
             x_bf, w_slab, send_buf, recv_buf,
             w_sem, send_sems, recv_sems):
        my = lax.axis_index("i")

        barrier = pltpu.get_barrier_semaphore()
        for d in range(1, N_DEV):
            pl.semaphore_signal(
                barrier, inc=1,
                device_id=((my + d) % N_DEV,),
                device_id_type=pl.DeviceIdType.MESH,
            )
        pl.semaphore_wait(barrier, N_DEV - 1)

        scale = sx_ref[0] * sw_ref[0]
        x_bf[:, :] = x_ref[:, :].astype(jnp.bfloat16)

        def block_for(dst):
            cp = pltpu.make_async_copy(
                w_ref.at[:, pl.ds(dst * n_per, n_per)], w_slab, w_sem)
            cp.start()
            cp.wait()
            acc = lax.dot_general(
                x_bf[:, :], w_slab[:, :].astype(jnp.bfloat16),
                (((1,), (0,)), ((), ())),
                preferred_element_type=jnp.float32,
            )
            y = acc * scale
            return y * (1.0 / (1.0 + jnp.exp(-y)))

        out_ref[pl.ds(my * m_per, m_per), :] = block_for(my)

        for d in range(1, N_DEV):
            dst = (my + d) % N_DEV
            src = (my + (N_DEV - d)) % N_DEV
            send_buf[d - 1, :, :] = block_for(dst).astype(jnp.bfloat16)
            rdma = pltpu.make_async_remote_copy(
                src_ref=send_buf.at[d - 1],
                dst_ref=recv_buf.at[d - 1],
                send_sem=send_sems.at[d - 1],
                recv_sem=recv_sems.at[d - 1],
                device_id=(dst,),
                device_id_type=pl.DeviceIdType.MESH,
            )
            rdma.start()
            rdma.wait()
            out_ref[pl.ds(src * m_per, m_per), :] = (
                recv_buf[d - 1, :, :].astype(jnp.float32))

    return pl.pallas_call(
        body,
        out_shape=jax.ShapeDtypeStruct((N_DEV * m_per, n_per), jnp.float32),
        in_specs=[
            pl.BlockSpec(memory_space=pltpu.MemorySpace.VMEM),
            pl.BlockSpec(memory_space=pltpu.MemorySpace.HBM),
            pl.BlockSpec(memory_space=pltpu.MemorySpace.SMEM),
            pl.BlockSpec(memory_space=pltpu.MemorySpace.SMEM),
        ],
        out_specs=pl.BlockSpec(memory_space=pltpu.MemorySpace.VMEM),
        scratch_shapes=[
            pltpu.VMEM((m_per, k), jnp.bfloat16),
            pltpu.VMEM((k, n_per), jnp.float32),
            pltpu.VMEM((N_DEV - 1, m_per, n_per), jnp.bfloat16),
            pltpu.VMEM((N_DEV - 1, m_per, n_per), jnp.bfloat16),
            pltpu.SemaphoreType.DMA,
            pltpu.SemaphoreType.DMA((N_DEV - 1,)),
            pltpu.SemaphoreType.DMA((N_DEV - 1,)),
        ],
        compiler_params=pltpu.CompilerParams(
            collective_id=0,
            vmem_limit_bytes=60 * 1024 * 1024,
        ),
    )(x, w_mat, scale_x, scale_w)


# device time: 56839 ns/iter; 1.6347x vs baseline; 1.6347x over previous
import jax
import jax.numpy as jnp
from jax import lax
from jax.experimental import pallas as pl
from jax.experimental.pallas import tpu as pltpu

N_DEV = 4


def kernel(x, w_mat, scale_x, scale_w):
    m_per, k = x.shape
    _, n = w_mat.shape
    n_per = n // N_DEV

    def body(x_ref, w_ref, sx_ref, sw_ref, out_ref,
             x_bf, w_slab, send_buf, recv_buf,
             w_sems, send_sems, recv_sems):
        my = lax.axis_index("i")

        def w_copy(dst, slot):
            return pltpu.make_async_copy(
                w_ref.at[:, pl.ds(dst * n_per, n_per)],
                w_slab.at[slot], w_sems.at[slot])

        w_copy((my + 1) % N_DEV, 0).start()

        barrier = pltpu.get_barrier_semaphore()
        for d in range(1, N_DEV):
            pl.semaphore_signal(
                barrier, inc=1,
                device_id=((my + d) % N_DEV,),
                device_id_type=pl.DeviceIdType.MESH,
            )
        pl.semaphore_wait(barrier, N_DEV - 1)

        scale = sx_ref[0] * sw_ref[0]
        x_bf[:, :] = x_ref[:, :].astype(jnp.bfloat16)

        def block_from_slab(slot):
            acc = lax.dot_general(
                x_bf[:, :], w_slab[slot, :, :].astype(jnp.bfloat16),
                (((1,), (0,)), ((), ())),
                preferred_element_type=jnp.float32,
            )
            y = acc * scale
            return y * (1.0 / (1.0 + jnp.exp(-y)))

        rdmas = []
        for d in range(1, N_DEV):
            dst = (my + d) % N_DEV
            slot = (d - 1) % 2
            nxt = (my + d + 1) % N_DEV
            w_copy(nxt, 1 - slot).start()
            w_copy(dst, slot).wait()
            send_buf[d - 1, :, :] = block_from_slab(slot).astype(jnp.bfloat16)
            rdma = pltpu.make_async_remote_copy(
                src_ref=send_buf.at[d - 1],
                dst_ref=recv_buf.at[d - 1],
                send_sem=send_sems.at[d - 1],
                recv_sem=recv_sems.at[d - 1],
                device_id=(dst,),
                device_id_type=pl.DeviceIdType.MESH,
            )
            rdma.start()
            rdmas.append(rdma)

        own_slot = 1 - ((N_DEV - 1 - 1) % 2)
        w_copy(my, own_slot).wait()
        out_ref[pl.ds(my * m_per, m_per), :] = block_from_slab(own_slot)

        for d in (1, 3, 2):
            src = (my + (N_DEV - d)) % N_DEV
            rdmas[d - 1].wait()
            out_ref[pl.ds(src * m_per, m_per), :] = (
                recv_buf[d - 1, :, :].astype(jnp.float32))

    return pl.pallas_call(
        body,
        out_shape=jax.ShapeDtypeStruct((N_DEV * m_per, n_per), jnp.float32),
        in_specs=[
            pl.BlockSpec(memory_space=pltpu.MemorySpace.VMEM),
            pl.BlockSpec(memory_space=pltpu.MemorySpace.HBM),
            pl.BlockSpec(memory_space=pltpu.MemorySpace.SMEM),
            pl.BlockSpec(memory_space=pltpu.MemorySpace.SMEM),
        ],
        out_specs=pl.BlockSpec(memory_space=pltpu.MemorySpace.VMEM),
        scratch_shapes=[
            pltpu.VMEM((m_per, k), jnp.bfloat16),
            pltpu.VMEM((2, k, n_per), jnp.float32),
            pltpu.VMEM((N_DEV - 1, m_per, n_per), jnp.bfloat16),
            pltpu.VMEM((N_DEV - 1, m_per, n_per), jnp.bfloat16),
            pltpu.SemaphoreType.DMA((2,)),
            pltpu.SemaphoreType.DMA((N_DEV - 1,)),
            pltpu.SemaphoreType.DMA((N_DEV - 1,)),
        ],
        compiler_params=pltpu.CompilerParams(
            collective_id=0,
            vmem_limit_bytes=62 * 1024 * 1024,
        ),
    )(x, w_mat, scale_x, scale_w)


# device time: 47457 ns/iter; 1.9579x vs baseline; 1.1977x over previous
import jax
import jax.numpy as jnp
from jax import lax
from jax.experimental import pallas as pl
from jax.experimental.pallas import tpu as pltpu

N_DEV = 4
SEND_ORDER = (2, 1, 3)
N_HALF = 2


def kernel(x, w_mat, scale_x, scale_w):
    m_per, k = x.shape
    _, n = w_mat.shape
    n_per = n // N_DEV
    m_h = m_per // N_HALF

    def body(x_ref, w_ref, sx_ref, sw_ref, out_ref,
             x_stage, x_f8, w_slab, w_f8, send_buf, recv_buf,
             x_sems, w_sems, send_sems, recv_sems):
        my = lax.axis_index("i")

        def w_copy(dst, slot):
            return pltpu.make_async_copy(
                w_ref.at[:, pl.ds(dst * n_per, n_per)],
                w_slab.at[slot], w_sems.at[slot])

        def x_copy(c):
            sl = pl.ds(c * m_h, m_h)
            return pltpu.make_async_copy(
                x_ref.at[sl], x_stage.at[sl], x_sems.at[c])

        with jax.named_scope("phase_entry_dma"):
            x_copy(0).start()
            w_copy((my + SEND_ORDER[0]) % N_DEV, 0).start()
            x_copy(1).start()

        with jax.named_scope("phase_barrier"):
            barrier = pltpu.get_barrier_semaphore()
            for d in range(1, N_DEV):
                pl.semaphore_signal(
                    barrier, inc=1,
                    device_id=((my + d) % N_DEV,),
                    device_id_type=pl.DeviceIdType.MESH,
                )
            pl.semaphore_wait(barrier, N_DEV - 1)

        scale = sx_ref[0] * sw_ref[0]

        def x_convert(c):
            sl = pl.ds(c * m_h, m_h)
            x_copy(c).wait()
            x_f8[sl, :] = x_stage[sl, :].astype(jnp.float8_e4m3fn)

        with jax.named_scope("phase_xconv0"):
            x_convert(0)

        def slab_convert(slot):
            w_f8[slot, :, :] = w_slab[slot, :, :].astype(jnp.float8_e4m3fn)

        def half_block(c, slot):
            acc = lax.dot_general(
                x_f8[pl.ds(c * m_h, m_h), :],
                w_f8[slot, :, :],
                (((1,), (0,)), ((), ())),
                preferred_element_type=jnp.float32,
            )
            y = acc * scale
            return y * (1.0 / (1.0 + jnp.exp(-y)))

        rdmas = {}
        for i, d in enumerate(SEND_ORDER):
            dst = (my + d) % N_DEV
            slot = i % 2
            nxt_d = SEND_ORDER[i + 1] if i + 1 < len(SEND_ORDER) else 0
            with jax.named_scope(f"phase_slabwait#d={d}"):
                w_copy((my + nxt_d) % N_DEV, 1 - slot).start()
                w_copy(dst, slot).wait()
                slab_convert(slot)
            with jax.named_scope(f"phase_blocksend#d={d}"):
                for c in range(N_HALF):
                    if i == 0 and c == 1:
                        x_convert(1)
                    sl = pl.ds(c * m_h, m_h)
                    send_buf[d - 1, sl, :] = half_block(c, slot).astype(jnp.bfloat16)
                    rdma = pltpu.make_async_remote_copy(
                        src_ref=send_buf.at[d - 1, sl],
                        dst_ref=recv_buf.at[d - 1, sl],
                        send_sem=send_sems.at[d - 1, c],
                        recv_sem=recv_sems.at[d - 1, c],
                        device_id=(dst,),
                        device_id_type=pl.DeviceIdType.MESH,
                    )
                    rdma.start()
                    rdmas[d, c] = rdma

        with jax.named_scope("phase_own"):
            own_slot = 1 - (len(SEND_ORDER) - 1) % 2
            w_copy(my, own_slot).wait()
            slab_convert(own_slot)
            for c in range(N_HALF):
                out_ref[pl.ds(my * m_per + c * m_h, m_h), :] = half_block(c, own_slot)

        for d in (1, 3, 2):
            src = (my + (N_DEV - d)) % N_DEV
            with jax.named_scope(f"phase_drain#d={d}"):
                for c in range(N_HALF):
                    rdmas[d, c].wait()
                    out_ref[pl.ds(src * m_per + c * m_h, m_h), :] = (
                        recv_buf[d - 1, pl.ds(c * m_h, m_h), :].astype(jnp.float32))

    return pl.pallas_call(
        body,
        out_shape=jax.ShapeDtypeStruct((N_DEV * m_per, n_per), jnp.float32),
        in_specs=[
            pl.BlockSpec(memory_space=pltpu.MemorySpace.HBM),
            pl.BlockSpec(memory_space=pltpu.MemorySpace.HBM),
            pl.BlockSpec(memory_space=pltpu.MemorySpace.SMEM),
            pl.BlockSpec(memory_space=pltpu.MemorySpace.SMEM),
        ],
        out_specs=pl.BlockSpec(memory_space=pltpu.MemorySpace.VMEM),
        scratch_shapes=[
            pltpu.VMEM((m_per, k), jnp.float32),
            pltpu.VMEM((m_per, k), jnp.float8_e4m3fn),
            pltpu.VMEM((2, k, n_per), jnp.float32),
            pltpu.VMEM((2, k, n_per), jnp.float8_e4m3fn),
            pltpu.VMEM((N_DEV - 1, m_per, n_per), jnp.bfloat16),
            pltpu.VMEM((N_DEV - 1, m_per, n_per), jnp.bfloat16),
            pltpu.SemaphoreType.DMA((N_HALF,)),
            pltpu.SemaphoreType.DMA((2,)),
            pltpu.SemaphoreType.DMA((N_DEV - 1, N_HALF)),
            pltpu.SemaphoreType.DMA((N_DEV - 1, N_HALF)),
        ],
        compiler_params=pltpu.CompilerParams(
            collective_id=0,
            vmem_limit_bytes=62 * 1024 * 1024,
        ),
    )(x, w_mat, scale_x, scale_w)


# device time: 45464 ns/iter; 2.0437x vs baseline; 1.0438x over previous
import jax
import jax.numpy as jnp
from jax import lax
from jax.experimental import pallas as pl
from jax.experimental.pallas import tpu as pltpu

N_DEV = 4
SEND_ORDER = (2, 1, 3)
N_HALF = 4


def kernel(x, w_mat, scale_x, scale_w):
    m_per, k = x.shape
    _, n = w_mat.shape
    n_per = n // N_DEV
    m_h = m_per // N_HALF

    def body(x_ref, w_ref, sx_ref, sw_ref, out_ref,
             x_stage, x_f8, w_slab, w_f8, send_buf, recv_buf, out_stage,
             x_sems, w_sems, send_sems, recv_sems, out_sems):
        pending = {}

        def store_block(idx, row0, fill):
            slot = idx % 2
            if slot in pending:
                pending[slot].wait()
            fill(out_stage.at[slot])
            cp = pltpu.make_async_copy(
                out_stage.at[slot],
                out_ref.at[pl.ds(row0, m_per)], out_sems.at[slot])
            cp.start()
            pending[slot] = cp
        my = lax.axis_index("i")

        def w_copy(dst, slot):
            return pltpu.make_async_copy(
                w_ref.at[:, pl.ds(dst * n_per, n_per)],
                w_slab.at[slot], w_sems.at[slot])

        def x_copy(c):
            sl = pl.ds(c * m_h, m_h)
            return pltpu.make_async_copy(
                x_ref.at[sl], x_stage.at[sl], x_sems.at[c])

        x_copy(0).start()
        w_copy((my + SEND_ORDER[0]) % N_DEV, 0).start()
        for c in range(1, N_HALF):
            x_copy(c).start()

        barrier = pltpu.get_barrier_semaphore()
        for d in range(1, N_DEV):
            pl.semaphore_signal(
                barrier, inc=1,
                device_id=((my + d) % N_DEV,),
                device_id_type=pl.DeviceIdType.MESH,
            )
        pl.semaphore_wait(barrier, N_DEV - 1)

        scale = sx_ref[0] * sw_ref[0]

        def x_convert(c):
            sl = pl.ds(c * m_h, m_h)
            x_copy(c).wait()
            x_f8[sl, :] = x_stage[sl, :].astype(jnp.float8_e4m3fn)

        x_convert(0)

        def slab_convert(slot):
            w_f8[slot, :, :] = w_slab[slot, :, :].astype(jnp.float8_e4m3fn)

        def half_block(c, slot):
            acc = lax.dot_general(
                x_f8[pl.ds(c * m_h, m_h), :],
                w_f8[slot, :, :],
                (((1,), (0,)), ((), ())),
                preferred_element_type=jnp.float32,
            )
            y = acc * scale
            return y * (1.0 / (1.0 + jnp.exp(-y)))

        rdmas = {}
        for i, d in enumerate(SEND_ORDER):
            dst = (my + d) % N_DEV
            slot = i % 2
            nxt_d = SEND_ORDER[i + 1] if i + 1 < len(SEND_ORDER) else 0
            w_copy((my + nxt_d) % N_DEV, 1 - slot).start()
            w_copy(dst, slot).wait()
            slab_convert(slot)
            for c in range(N_HALF):
                if i == 0 and c > 0:
                    x_convert(c)
                sl = pl.ds(c * m_h, m_h)
                send_buf[d - 1, sl, :] = half_block(c, slot).astype(jnp.bfloat16)
                rdma = pltpu.make_async_remote_copy(
                    src_ref=send_buf.at[d - 1, sl],
                    dst_ref=recv_buf.at[d - 1, sl],
                    send_sem=send_sems.at[d - 1, c],
                    recv_sem=recv_sems.at[d - 1, c],
                    device_id=(dst,),
                    device_id_type=pl.DeviceIdType.MESH,
                )
                rdma.start()
                rdmas[d, c] = rdma

        own_slot = 1 - (len(SEND_ORDER) - 1) % 2
        w_copy(my, own_slot).wait()
        slab_convert(own_slot)

        def fill_own(stage):
            for c in range(N_HALF):
                stage[pl.ds(c * m_h, m_h), :] = half_block(c, own_slot)
        store_block(0, my * m_per, fill_own)

        for k, d in enumerate((1, 3, 2)):
            src = (my + (N_DEV - d)) % N_DEV

            def fill_recv(stage, d=d):
                for c in range(N_HALF):
                    rdmas[d, c].wait()
                    stage[pl.ds(c * m_h, m_h), :] = (
                        recv_buf[d - 1, pl.ds(c * m_h, m_h), :]
                        .astype(jnp.float32))
            store_block(k + 1, src * m_per, fill_recv)

        for cp in pending.values():
            cp.wait()

    return pl.pallas_call(
        body,
        out_shape=jax.ShapeDtypeStruct((N_DEV * m_per, n_per), jnp.float32),
        in_specs=[
            pl.BlockSpec(memory_space=pltpu.MemorySpace.HBM),
            pl.BlockSpec(memory_space=pltpu.MemorySpace.HBM),
            pl.BlockSpec(memory_space=pltpu.MemorySpace.SMEM),
            pl.BlockSpec(memory_space=pltpu.MemorySpace.SMEM),
        ],
        out_specs=pl.BlockSpec(memory_space=pltpu.MemorySpace.HBM),
        scratch_shapes=[
            pltpu.VMEM((m_per, k), jnp.float32),
            pltpu.VMEM((m_per, k), jnp.float8_e4m3fn),
            pltpu.VMEM((2, k, n_per), jnp.float32),
            pltpu.VMEM((2, k, n_per), jnp.float8_e4m3fn),
            pltpu.VMEM((N_DEV - 1, m_per, n_per), jnp.bfloat16),
            pltpu.VMEM((N_DEV - 1, m_per, n_per), jnp.bfloat16),
            pltpu.VMEM((2, m_per, n_per), jnp.float32),
            pltpu.SemaphoreType.DMA((N_HALF,)),
            pltpu.SemaphoreType.DMA((2,)),
            pltpu.SemaphoreType.DMA((N_DEV - 1, N_HALF)),
            pltpu.SemaphoreType.DMA((N_DEV - 1, N_HALF)),
            pltpu.SemaphoreType.DMA((2,)),
        ],
        compiler_params=pltpu.CompilerParams(
            collective_id=0,
            vmem_limit_bytes=62 * 1024 * 1024,
        ),
    )(x, w_mat, scale_x, scale_w)
